# Initial kernel scaffold; baseline (speedup 1.0000x reference)
import jax
import jax.numpy as jnp
from jax.experimental import pallas as pl


def kernel(gt_boxes, anchor_boxes, unit_lengths, min_anchor_size):
    raise NotImplementedError("write your pallas kernel here")


# speedup vs baseline: 2.9930x; 2.9930x over previous
import jax
import jax.numpy as jnp
from jax import lax
from jax.experimental import pallas as pl

_BLOCK = 2048


def _match_block(gt_ref, anc_ref, u_ref, mas_ref, match_ref, cnt_ref):
    gt = gt_ref[...]
    gx0 = gt[:, 0:1]
    gy0 = gt[:, 1:2]
    gx1 = gt[:, 2:3]
    gy1 = gt[:, 3:4]
    anc = anc_ref[...]
    ax0 = anc[0:1, :]
    ay0 = anc[1:2, :]
    ax1 = anc[2:3, :]
    ay1 = anc[3:4, :]
    u = u_ref[...]
    mas = mas_ref[0, 0]

    contain = (gx0 >= ax0) & (gy0 >= ay0) & (gx1 <= ax1) & (gy1 <= ay1)

    gt_size = jnp.maximum(gx1 - gx0, gy1 - gy0)
    gt_upper = gt_size * 2.0
    gt_upper = jnp.where(gt_upper < mas, mas, gt_upper)
    an_size = jnp.maximum(ax1 - ax0, ay1 - ay0) - u
    scale = (gt_upper - an_size) >= 0.0

    dx = ((gx0 + gx1) / 2.0 - (ax0 + ax1) / 2.0) / u
    dy = ((gy0 + gy1) / 2.0 - (ay0 + ay1) / 2.0) / u
    spatial = (dx * dx + dy * dy) <= 1.0

    assign = contain & scale & spatial
    n = gt.shape[0]
    cnt = jnp.sum(assign.astype(jnp.int32), axis=0, keepdims=True)
    iota = lax.broadcasted_iota(jnp.int32, assign.shape, 0)
    first = jnp.min(jnp.where(assign, iota, n), axis=0, keepdims=True)
    match_ref[...] = jnp.where(cnt > 0, first, 0)
    cnt_ref[...] = cnt


def kernel(gt_boxes, anchor_boxes, unit_lengths, min_anchor_size):
    n = gt_boxes.shape[0]
    m = anchor_boxes.shape[0]
    anc_t = anchor_boxes.T
    u2 = unit_lengths.reshape(1, m)
    mas = jnp.asarray(min_anchor_size, jnp.float32).reshape(1, 1)
    matches2, cnt2 = pl.pallas_call(
        _match_block,
        grid=(pl.cdiv(m, _BLOCK),),
        in_specs=[
            pl.BlockSpec((n, 4), lambda j: (0, 0)),
            pl.BlockSpec((4, _BLOCK), lambda j: (0, j)),
            pl.BlockSpec((1, _BLOCK), lambda j: (0, j)),
            pl.BlockSpec((1, 1), lambda j: (0, 0)),
        ],
        out_specs=[
            pl.BlockSpec((1, _BLOCK), lambda j: (0, j)),
            pl.BlockSpec((1, _BLOCK), lambda j: (0, j)),
        ],
        out_shape=[
            jax.ShapeDtypeStruct((1, m), jnp.int32),
            jax.ShapeDtypeStruct((1, m), jnp.int32),
        ],
    )(gt_boxes, anc_t, u2, mas)
    matches = matches2.reshape(m)
    match_labels = (cnt2.reshape(m) == 1).astype(jnp.int8)
    return (matches, match_labels)


# speedup vs baseline: 3.0147x; 1.0073x over previous
import jax
import jax.numpy as jnp
from jax import lax
from jax.experimental import pallas as pl

_BLOCK = 2048


def _match_block(gt_ref, anc_ref, u_ref, mas_ref, match_ref, cnt_ref):
    gt = gt_ref[...]
    gx0 = gt[:, 0:1]
    gy0 = gt[:, 1:2]
    gx1 = gt[:, 2:3]
    gy1 = gt[:, 3:4]
    anc = anc_ref[...]
    ax0 = anc[0:1, :]
    ay0 = anc[1:2, :]
    ax1 = anc[2:3, :]
    ay1 = anc[3:4, :]
    u = u_ref[...]
    mas = mas_ref[0, 0]

    contain = (gx0 >= ax0) & (gy0 >= ay0) & (gx1 <= ax1) & (gy1 <= ay1)

    gt_size = jnp.maximum(gx1 - gx0, gy1 - gy0)
    gt_upper = gt_size * 2.0
    gt_upper = jnp.where(gt_upper < mas, mas, gt_upper)
    an_size = jnp.maximum(ax1 - ax0, ay1 - ay0) - u
    scale = (gt_upper - an_size) >= 0.0

    dx = (gx0 + gx1) / 2.0 - (ax0 + ax1) / 2.0
    dy = (gy0 + gy1) / 2.0 - (ay0 + ay1) / 2.0
    spatial = (dx * dx + dy * dy) <= u * u

    assign = contain & scale & spatial
    n = gt.shape[0]
    cnt = jnp.sum(assign.astype(jnp.int32), axis=0, keepdims=True)
    iota = lax.broadcasted_iota(jnp.int32, assign.shape, 0)
    first = jnp.min(jnp.where(assign, iota, n), axis=0, keepdims=True)
    match_ref[...] = jnp.where(cnt > 0, first, 0)
    cnt_ref[...] = cnt


def kernel(gt_boxes, anchor_boxes, unit_lengths, min_anchor_size):
    n = gt_boxes.shape[0]
    m = anchor_boxes.shape[0]
    anc_t = anchor_boxes.T
    u2 = unit_lengths.reshape(1, m)
    mas = jnp.asarray(min_anchor_size, jnp.float32).reshape(1, 1)
    matches2, cnt2 = pl.pallas_call(
        _match_block,
        grid=(pl.cdiv(m, _BLOCK),),
        in_specs=[
            pl.BlockSpec((n, 4), lambda j: (0, 0)),
            pl.BlockSpec((4, _BLOCK), lambda j: (0, j)),
            pl.BlockSpec((1, _BLOCK), lambda j: (0, j)),
            pl.BlockSpec((1, 1), lambda j: (0, 0)),
        ],
        out_specs=[
            pl.BlockSpec((1, _BLOCK), lambda j: (0, j)),
            pl.BlockSpec((1, _BLOCK), lambda j: (0, j)),
        ],
        out_shape=[
            jax.ShapeDtypeStruct((1, m), jnp.int32),
            jax.ShapeDtypeStruct((1, m), jnp.int32),
        ],
    )(gt_boxes, anc_t, u2, mas)
    matches = matches2.reshape(m)
    match_labels = (cnt2.reshape(m) == 1).astype(jnp.int8)
    return (matches, match_labels)


# speedup vs baseline: 3.8043x; 1.2619x over previous
import jax
import jax.numpy as jnp
from jax import lax
from jax.experimental import pallas as pl

_BLOCK = 2048


def _match_block(gt_ref, anc_ref, u_ref, mas_ref, match_ref, cnt_ref):
    gt = gt_ref[...]
    gx0 = gt[:, 0:1]
    gy0 = gt[:, 1:2]
    gx1 = gt[:, 2:3]
    gy1 = gt[:, 3:4]
    anc = anc_ref[...]
    ax0 = anc[0:1, :]
    ay0 = anc[1:2, :]
    ax1 = anc[2:3, :]
    ay1 = anc[3:4, :]
    u = u_ref[...]
    mas = mas_ref[0, 0]

    gt_upper = jnp.maximum(gx1 - gx0, gy1 - gy0) * 2.0
    gt_upper = jnp.where(gt_upper < mas, mas, gt_upper)
    gcx = (gx0 + gx1) / 2.0
    gcy = (gy0 + gy1) / 2.0
    an_size = jnp.maximum(ax1 - ax0, ay1 - ay0) - u
    acx = (ax0 + ax1) / 2.0
    acy = (ay0 + ay1) / 2.0
    uu = u * u

    m = jnp.minimum(gx0 - ax0, gy0 - ay0)
    m = jnp.minimum(m, ax1 - gx1)
    m = jnp.minimum(m, ay1 - gy1)
    m = jnp.minimum(m, gt_upper - an_size)
    dx = gcx - acx
    dy = gcy - acy
    m = jnp.minimum(m, uu - (dx * dx + dy * dy))
    assign = m >= 0.0

    n = gt.shape[0]
    cnt = jnp.sum(assign.astype(jnp.int32), axis=0, keepdims=True)
    iota = lax.broadcasted_iota(jnp.int32, assign.shape, 0)
    first = jnp.min(jnp.where(assign, iota, n), axis=0, keepdims=True)
    match_ref[...] = jnp.where(cnt > 0, first, 0)
    cnt_ref[...] = cnt


def kernel(gt_boxes, anchor_boxes, unit_lengths, min_anchor_size):
    n = gt_boxes.shape[0]
    m = anchor_boxes.shape[0]
    anc_t = anchor_boxes.T
    u2 = unit_lengths.reshape(1, m)
    mas = jnp.asarray(min_anchor_size, jnp.float32).reshape(1, 1)
    matches2, cnt2 = pl.pallas_call(
        _match_block,
        grid=(pl.cdiv(m, _BLOCK),),
        in_specs=[
            pl.BlockSpec((n, 4), lambda j: (0, 0)),
            pl.BlockSpec((4, _BLOCK), lambda j: (0, j)),
            pl.BlockSpec((1, _BLOCK), lambda j: (0, j)),
            pl.BlockSpec((1, 1), lambda j: (0, 0)),
        ],
        out_specs=[
            pl.BlockSpec((1, _BLOCK), lambda j: (0, j)),
            pl.BlockSpec((1, _BLOCK), lambda j: (0, j)),
        ],
        out_shape=[
            jax.ShapeDtypeStruct((1, m), jnp.int32),
            jax.ShapeDtypeStruct((1, m), jnp.int32),
        ],
    )(gt_boxes, anc_t, u2, mas)
    matches = matches2.reshape(m)
    match_labels = (cnt2.reshape(m) == 1).astype(jnp.int8)
    return (matches, match_labels)
